# baseline (speedup 1.0000x reference)
import jax
import jax.numpy as jnp
from jax import lax
from jax.experimental import pallas as pl
from jax.experimental.pallas import tpu as pltpu
from jax.experimental.pallas import tpu_sc as plsc

B, S = 4, 8192
D = 1024
NC, NS = 2, 16
NW = NC * NS
TOTAL = B * S
PER_W = TOTAL // NW
CHUNK = 32
NCH = PER_W // CHUNK
NSLOT = 3
NG = 10
GOOD = 0x01010101


def _body(x_hbm, pe_hbm, out_hbm, idx_v, rows_v, g0, g1, g2, o0, o1, o2):
    gsem = (g0, g1, g2)
    osem = (o0, o1, o2)
    wid = lax.axis_index("s") * NC + lax.axis_index("c")
    base = wid * PER_W

    pltpu.sync_copy(x_hbm.at[wid], idx_v)

    zeros = jnp.zeros((16,), jnp.float32)

    def slot(s):
        return rows_v.at[pl.ds(s * CHUNK, CHUNK)]

    def gather(c, s):
        pltpu.make_async_copy(pe_hbm.at[idx_v.at[c]], slot(s), gsem[s]).start()

    def gather_wait(c, s):
        pltpu.make_async_copy(pe_hbm.at[idx_v.at[c]], slot(s), gsem[s]).wait()

    def out_start(c, s):
        dst = out_hbm.at[pl.ds(base + c * CHUNK, CHUNK)]
        pltpu.make_async_copy(slot(s), dst, osem[s]).start()

    def out_wait(s):
        dst = out_hbm.at[pl.ds(base, CHUNK)]
        pltpu.make_async_copy(slot(s), dst, osem[s]).wait()

    gather(0, 0)
    gather(1, 1)

    def mask_body(k, m):
        zs = [jnp.minimum(idx_v[2 * k, pl.ds(0, 16)], 1),
              jnp.minimum(idx_v[2 * k, pl.ds(16, 16)], 1),
              jnp.minimum(idx_v[2 * k + 1, pl.ds(0, 16)], 1),
              jnp.minimum(idx_v[2 * k + 1, pl.ds(16, 16)], 1)]
        packed = zs[0] | (zs[1] << 8) | (zs[2] << 16) | (zs[3] << 24)
        flag = packed[0] != GOOD
        for l in range(1, 16):
            flag = jnp.logical_or(flag, packed[l] != GOOD)
        return m | (jnp.where(flag, 1, 0) << k)

    zmask = lax.fori_loop(0, NCH // 2, mask_body, 0)

    def fixup(c, s):
        @pl.when(((zmask >> (c // 2)) & 1) != 0)
        def _slow():
            for h in range(2):
                vec = idx_v[c, pl.ds(h * 16, 16)]
                for r in range(16):
                    @pl.when(vec[r] == 0)
                    def _zero_row():
                        def zb(j, carry):
                            rows_v[s * CHUNK + h * 16 + r,
                                   pl.ds(j * 16, 16)] = zeros
                            return carry

                        lax.fori_loop(0, D // 16, zb, 0)

    def chunk_step(c, s, issue=True, skip_wait_if=None):
        gather_wait(c, s)
        fixup(c, s)
        out_start(c, s)
        if issue:
            st = (s + 2) % NSLOT
            if skip_wait_if is None:
                out_wait(st)
            else:
                @pl.when(jnp.logical_not(skip_wait_if))
                def _w():
                    out_wait(st)

            gather(c + 2, st)

    def outer(g, carry):
        for j in range(NSLOT):
            c = g * NSLOT + j
            chunk_step(c, j, skip_wait_if=(g == 0) if j == 0 else None)
        return carry

    lax.fori_loop(0, NG, outer, 0)

    chunk_step(30, 0, issue=False)
    chunk_step(31, 1, issue=False)

    out_wait(2)
    out_wait(0)
    out_wait(1)


@jax.jit
def _sc_embed(x_r, pe):
    mesh = plsc.VectorSubcoreMesh(core_axis_name="c", subcore_axis_name="s")
    return pl.kernel(
        _body,
        out_type=jax.ShapeDtypeStruct((TOTAL, D), jnp.float32),
        mesh=mesh,
        scratch_types=[
            pltpu.VMEM((NCH, CHUNK), jnp.int32),
            pltpu.VMEM((NSLOT * CHUNK, D), jnp.float32),
            pltpu.SemaphoreType.DMA,
            pltpu.SemaphoreType.DMA,
            pltpu.SemaphoreType.DMA,
            pltpu.SemaphoreType.DMA,
            pltpu.SemaphoreType.DMA,
            pltpu.SemaphoreType.DMA,
        ],
    )(x_r, pe)


def kernel(x, pe):
    out = _sc_embed(x.reshape(NW, NCH, CHUNK), pe)
    return out.reshape(B, S, D)


# speedup vs baseline: 1.0315x; 1.0315x over previous
import jax
import jax.numpy as jnp
from jax import lax
from jax.experimental import pallas as pl
from jax.experimental.pallas import tpu as pltpu
from jax.experimental.pallas import tpu_sc as plsc

B, S = 4, 8192
D = 1024
NC, NS = 2, 16
NW = NC * NS
TOTAL = B * S
PER_W = TOTAL // NW
CHUNK = 16
NCH = PER_W // CHUNK
NBUF = 4
NG = NCH // NBUF


def _body(x_hbm, pe_hbm, out_hbm, idx_v,
          rows0, rows1, rows2, rows3,
          g0, g1, g2, g3, o0, o1, o2, o3):
    rows = (rows0, rows1, rows2, rows3)
    gsem = (g0, g1, g2, g3)
    osem = (o0, o1, o2, o3)
    wid = lax.axis_index("s") * NC + lax.axis_index("c")
    base = wid * PER_W

    pltpu.sync_copy(x_hbm.at[wid], idx_v)

    zeros = jnp.zeros((16,), jnp.float32)

    def gather(c, b):
        pltpu.make_async_copy(pe_hbm.at[idx_v.at[c]], rows[b], gsem[b]).start()

    def gather_wait(c, b):
        pltpu.make_async_copy(pe_hbm.at[idx_v.at[c]], rows[b], gsem[b]).wait()

    GOOD = 0x01010101

    def group_detect(g):
        zs = [jnp.minimum(idx_v[g * NBUF + b, pl.ds(0, 16)], 1)
              for b in range(NBUF)]
        packed = zs[0] | (zs[1] << 8) | (zs[2] << 16) | (zs[3] << 24)
        svals = [packed[l] for l in range(16)]
        flag = svals[0] != GOOD
        for l in range(1, 16):
            flag = jnp.logical_or(flag, svals[l] != GOOD)
        return flag, svals

    def fixup(b, gflag, svals):
        @pl.when(gflag)
        def _slow():
            for r in range(CHUNK):
                @pl.when(((svals[r] >> (8 * b)) & 0xFF) == 0)
                def _zero_row():
                    def zb(j, carry):
                        rows[b][r, pl.ds(j * 16, 16)] = zeros
                        return carry

                    lax.fori_loop(0, D // 16, zb, 0)

    def out_start(c, b):
        dst = out_hbm.at[pl.ds(base + c * CHUNK, CHUNK)]
        pltpu.make_async_copy(rows[b], dst, osem[b]).start()

    def out_wait(b):
        dst = out_hbm.at[pl.ds(base, CHUNK)]
        pltpu.make_async_copy(rows[b], dst, osem[b]).wait()

    for b in range(NBUF - 1):
        gather(b, b)

    def outer(g, carry):
        gflag, svals = group_detect(g)
        for b in range(NBUF):
            c = g * NBUF + b
            gather_wait(c, b)
            fixup(b, gflag, svals)
            out_start(c, b)
            bt = (b + NBUF - 1) % NBUF
            if b == 0:
                @pl.when(g > 0)
                def _w0():
                    out_wait(bt)

                gather(c + NBUF - 1, bt)
            else:
                @pl.when(g < NG - 1)
                def _wn():
                    out_wait(bt)
                    gather(c + NBUF - 1, bt)

        return carry

    lax.fori_loop(0, NG, outer, 0)

    for b in range(NBUF):
        out_wait(b)


@jax.jit
def _sc_embed(x_r, pe):
    mesh = plsc.VectorSubcoreMesh(core_axis_name="c", subcore_axis_name="s")
    return pl.kernel(
        _body,
        out_type=jax.ShapeDtypeStruct((TOTAL, D), jnp.float32),
        mesh=mesh,
        scratch_types=[
            pltpu.VMEM((NCH, CHUNK), jnp.int32),
            pltpu.VMEM((CHUNK, D), jnp.float32),
            pltpu.VMEM((CHUNK, D), jnp.float32),
            pltpu.VMEM((CHUNK, D), jnp.float32),
            pltpu.VMEM((CHUNK, D), jnp.float32),
            pltpu.SemaphoreType.DMA,
            pltpu.SemaphoreType.DMA,
            pltpu.SemaphoreType.DMA,
            pltpu.SemaphoreType.DMA,
            pltpu.SemaphoreType.DMA,
            pltpu.SemaphoreType.DMA,
            pltpu.SemaphoreType.DMA,
            pltpu.SemaphoreType.DMA,
        ],
    )(x_r, pe)


def kernel(x, pe):
    out = _sc_embed(x.reshape(NW, NCH, CHUNK), pe)
    return out.reshape(B, S, D)
